# baseline (speedup 1.0000x reference)
import functools

import jax
import jax.numpy as jnp
from jax import lax
from jax.experimental import pallas as pl
from jax.experimental.pallas import tpu as pltpu
from jax.experimental.pallas import tpu_sc as plsc

B = 16384
A = 5
H = 64
NC = 2
NS = 16
NW = NC * NS
BPW = B // NW
IDX_ROWS = BPW // 128
L = 16


def _bias_lookup_body(uid_hbm, iid_hbm, utab_hbm, itab_hbm, out_hbm,
                      uid_v, iid_v, ub_v, ib_v, sem_u, sem_i):
    wid = lax.axis_index("s") * NC + lax.axis_index("c")
    rbase = wid * IDX_ROWS
    pltpu.sync_copy(uid_hbm.at[pl.ds(rbase, IDX_ROWS)], uid_v)
    pltpu.sync_copy(iid_hbm.at[pl.ds(rbase, IDX_ROWS)], iid_v)
    utab = utab_hbm.at[0]
    itab = itab_hbm.at[0]
    copies = []
    for r in range(IDX_ROWS):
        copies.append(pltpu.async_copy(utab.at[uid_v.at[r]], ub_v.at[r], sem_u))
        copies.append(pltpu.async_copy(itab.at[iid_v.at[r]], ib_v.at[r], sem_i))
    for c in copies:
        c.wait()
    for r in range(IDX_ROWS):
        for k in range(128 // L):
            sl = pl.ds(k * L, L)
            ub_v[r, sl] = ub_v[r, sl] + ib_v[r, sl]
    pltpu.sync_copy(ub_v, out_hbm.at[pl.ds(rbase, IDX_ROWS)])


@functools.lru_cache(maxsize=1)
def _bias_lookup():
    mesh = plsc.VectorSubcoreMesh(core_axis_name="c", subcore_axis_name="s")
    return pl.kernel(
        _bias_lookup_body,
        out_type=jax.ShapeDtypeStruct((NW * IDX_ROWS, 128), jnp.float32),
        mesh=mesh,
        scratch_types=[
            pltpu.VMEM((IDX_ROWS, 128), jnp.int32),
            pltpu.VMEM((IDX_ROWS, 128), jnp.int32),
            pltpu.VMEM((IDX_ROWS, 128), jnp.float32),
            pltpu.VMEM((IDX_ROWS, 128), jnp.float32),
            pltpu.SemaphoreType.DMA,
            pltpu.SemaphoreType.DMA,
        ],
        compiler_params=pltpu.CompilerParams(use_tc_tiling_on_sc=False),
    )


def _dense_body(u_ref, i_ref, uw_ref, iw_ref, o_ref):
    acc = None
    for a in range(A):
        sl = pl.ds(a * H, H)
        pa = u_ref[:, sl] * i_ref[:, sl]
        term = jnp.sum(pa, axis=1, keepdims=True) * (
            uw_ref[:, a:a + 1] * iw_ref[:, a:a + 1])
        acc = term if acc is None else acc + term
    o_ref[...] = acc


def kernel(userAspRep, itemAspRep, userAspImpt, itemAspImpt, batch_uid,
           batch_iid, user_offset_table, item_offset_table, global_offset):
    u2 = userAspRep.reshape(B, A * H)
    i2 = itemAspRep.reshape(B, A * H)

    BB = 2048
    grid = B // BB
    dense = pl.pallas_call(
        _dense_body,
        grid=(grid,),
        in_specs=[
            pl.BlockSpec((BB, A * H), lambda b: (b, 0)),
            pl.BlockSpec((BB, A * H), lambda b: (b, 0)),
            pl.BlockSpec((BB, A), lambda b: (b, 0)),
            pl.BlockSpec((BB, A), lambda b: (b, 0)),
        ],
        out_specs=pl.BlockSpec((BB, 1), lambda b: (b, 0)),
        out_shape=jax.ShapeDtypeStruct((B, 1), jnp.float32),
    )(u2, i2, userAspImpt, itemAspImpt)

    bias = _bias_lookup()(
        batch_uid.astype(jnp.int32).reshape(B // 128, 128),
        batch_iid.astype(jnp.int32).reshape(B // 128, 128),
        user_offset_table.reshape(1, -1),
        item_offset_table.reshape(1, -1))

    return dense + bias.reshape(B, 1) + global_offset
